# baseline (speedup 1.0000x reference)
import numpy as np
import jax
import jax.numpy as jnp
from jax.experimental import pallas as pl

_HID = 128
_HEADS = 4
_DPH = _HID // _HEADS
_NT = 4
_ET = 38

_EB = 1024
_NC = 1024
_NPAD = 10240
_CPAD = 640
_NCHUNKS = _NPAD // _NC


def _scores_kernel(src_ref, dst_ref, cmb_ref, q_ref, k_ref, ke_ref,
                   sc_ref, smax_ref):
    @pl.when(pl.program_id(0) == 0)
    def _init():
        smax_ref[...] = jnp.full(smax_ref.shape, -jnp.inf, jnp.float32)

    src = src_ref[...]
    dst = dst_ref[...]
    cmb = cmb_ref[...]

    qs = jnp.zeros((_EB, _HID), jnp.float32)
    kd = jnp.zeros((_EB, _HID), jnp.float32)
    for c in range(_NCHUNKS):
        col = jax.lax.broadcasted_iota(jnp.int32, (_EB, _NC), 1) + c * _NC
        ms = (src == col).astype(jnp.float32)
        md = (dst == col).astype(jnp.float32)
        qs = qs + jax.lax.dot_general(
            ms, q_ref[c * _NC:(c + 1) * _NC, :],
            (((1,), (0,)), ((), ())), preferred_element_type=jnp.float32)
        kd = kd + jax.lax.dot_general(
            md, k_ref[c * _NC:(c + 1) * _NC, :],
            (((1,), (0,)), ((), ())), preferred_element_type=jnp.float32)
    ccol = jax.lax.broadcasted_iota(jnp.int32, (_EB, _CPAD), 1)
    mc = (cmb == ccol).astype(jnp.float32)
    kec = jax.lax.dot_general(
        mc, ke_ref[...], (((1,), (0,)), ((), ())),
        preferred_element_type=jnp.float32)

    prod = qs * (kd + kec)
    sc = jnp.concatenate(
        [jnp.sum(prod[:, h * _DPH:(h + 1) * _DPH], axis=1, keepdims=True)
         for h in range(_HEADS)], axis=1)
    sc_ref[...] = sc

    for c in range(_NCHUNKS):
        col = jax.lax.broadcasted_iota(jnp.int32, (_EB, _NC), 1) + c * _NC
        msb = (src == col)
        for h in range(_HEADS):
            v = jnp.where(msb, sc[:, h:h + 1], -jnp.inf)
            u = jnp.max(v, axis=0, keepdims=True)
            cur = smax_ref[h:h + 1, c * _NC:(c + 1) * _NC]
            smax_ref[h:h + 1, c * _NC:(c + 1) * _NC] = jnp.maximum(cur, u)


def _softmax_kernel(src_ref, sc_ref, smax_ref, ex_ref, den_ref):
    @pl.when(pl.program_id(0) == 0)
    def _init():
        den_ref[...] = jnp.zeros(den_ref.shape, jnp.float32)

    src = src_ref[...]
    sc = sc_ref[...]
    smg = [jnp.zeros((_EB, 1), jnp.float32) for _ in range(_HEADS)]
    for c in range(_NCHUNKS):
        col = jax.lax.broadcasted_iota(jnp.int32, (_EB, _NC), 1) + c * _NC
        ms = (src == col).astype(jnp.float32)
        for h in range(_HEADS):
            row = smax_ref[h:h + 1, c * _NC:(c + 1) * _NC]
            smg[h] = smg[h] + jnp.sum(ms * row, axis=1, keepdims=True)
    smaxg = jnp.concatenate(smg, axis=1)
    ex = jnp.exp(sc - smaxg)
    ex_ref[...] = ex
    for c in range(_NCHUNKS):
        col = jax.lax.broadcasted_iota(jnp.int32, (_EB, _NC), 1) + c * _NC
        ms = (src == col).astype(jnp.float32)
        for h in range(_HEADS):
            contrib = jnp.sum(ms * ex[:, h:h + 1], axis=0, keepdims=True)
            den_ref[h:h + 1, c * _NC:(c + 1) * _NC] = (
                den_ref[h:h + 1, c * _NC:(c + 1) * _NC] + contrib)
        cnt = jnp.sum(ms, axis=0, keepdims=True)
        den_ref[4:5, c * _NC:(c + 1) * _NC] = (
            den_ref[4:5, c * _NC:(c + 1) * _NC] + cnt)


def _aggr_kernel(src_ref, dstr_ref, cmb_ref, ex_ref, den_ref, m_ref, me_ref,
                 out_ref):
    @pl.when(pl.program_id(0) == 0)
    def _init():
        out_ref[...] = jnp.zeros(out_ref.shape, jnp.float32)

    src = src_ref[...]
    dstr = dstr_ref[...]
    cmb = cmb_ref[...]
    ex = ex_ref[...]

    msrc = jnp.zeros((_EB, _HID), jnp.float32)
    dg = [jnp.zeros((_EB, 1), jnp.float32) for _ in range(5)]
    for c in range(_NCHUNKS):
        col = jax.lax.broadcasted_iota(jnp.int32, (_EB, _NC), 1) + c * _NC
        ms = (src == col).astype(jnp.float32)
        msrc = msrc + jax.lax.dot_general(
            ms, m_ref[c * _NC:(c + 1) * _NC, :],
            (((1,), (0,)), ((), ())), preferred_element_type=jnp.float32)
        for j in range(5):
            row = den_ref[j:j + 1, c * _NC:(c + 1) * _NC]
            dg[j] = dg[j] + jnp.sum(ms * row, axis=1, keepdims=True)
    deng = jnp.concatenate(dg[:4], axis=1)
    cntg = dg[4]
    alpha = ex / (deng + 1e-16) * cntg

    ccol = jax.lax.broadcasted_iota(jnp.int32, (_EB, _CPAD), 1)
    mc = (cmb == ccol).astype(jnp.float32)
    mec = jax.lax.dot_general(
        mc, me_ref[...], (((1,), (0,)), ((), ())),
        preferred_element_type=jnp.float32)

    r = jax.lax.broadcasted_iota(jnp.int32, (_HEADS, _HID), 0)
    l = jax.lax.broadcasted_iota(jnp.int32, (_HEADS, _HID), 1) // _DPH
    he = (r == l).astype(jnp.float32)
    alpha128 = jax.lax.dot_general(
        alpha, he, (((1,), (0,)), ((), ())),
        preferred_element_type=jnp.float32)
    m = (msrc + mec) * alpha128

    for c in range(_NCHUNKS):
        rowi = jax.lax.broadcasted_iota(jnp.int32, (_NC, _EB), 0) + c * _NC
        mdT = (rowi == dstr).astype(jnp.float32)
        acc = jax.lax.dot_general(
            mdT, m, (((1,), (0,)), ((), ())),
            preferred_element_type=jnp.float32)
        out_ref[c * _NC:(c + 1) * _NC, :] = (
            out_ref[c * _NC:(c + 1) * _NC, :] + acc)


def _gelu(x):
    return jax.nn.gelu(x, approximate=True)


def _pad_rows(x, n):
    return jnp.pad(x, ((0, n - x.shape[0]), (0, 0)))


def kernel(H, node_score, params, edge_index, edge_type, node_type):
    p = params
    bs, nn_ = node_type.shape
    N_ = bs * nn_
    E = edge_index.shape[1]
    E_tot = E + N_
    nblk = -(-E_tot // _EB)
    E_pad = nblk * _EB

    T = jax.nn.one_hot(node_type, _NT, dtype=jnp.float32)
    node_type_emb = _gelu(T @ p['ent_W'].T + p['ent_b'])
    js = jnp.power(1.1, jnp.arange(_HID // 2, dtype=jnp.float32))[None, None, :]
    node_score_emb = _gelu(jnp.sin(js * node_score) @ p['esc_W'].T + p['esc_b'])
    X = H.reshape(N_, -1)
    nt_flat = node_type.reshape(-1)
    nfe = jnp.concatenate([node_type_emb, node_score_emb], axis=2).reshape(N_, -1)

    cid = jnp.arange(_ET + 1, dtype=jnp.int32)
    cc = (cid[:, None, None] * 16 + jnp.arange(_NT, dtype=jnp.int32)[None, :, None] * 4
          + jnp.arange(_NT, dtype=jnp.int32)[None, None, :]).reshape(-1)
    ev = jax.nn.one_hot(cc // 16, _ET + 1, dtype=jnp.float32)
    ha = jax.nn.one_hot((cc // 4) % 4, _NT, dtype=jnp.float32)
    hb = jax.nn.one_hot(cc % 4, _NT, dtype=jnp.float32)
    feat = jnp.concatenate([ev, ha, hb], axis=1)
    h1 = feat @ p['ee_W1'].T + p['ee_b1']
    h1 = (h1 / np.sqrt(1.0 + 1e-5)) * p['ee_g'] + p['ee_be']
    tab = jax.nn.relu(h1) @ p['ee_W2'].T + p['ee_b2']
    tab_p = _pad_rows(tab, _CPAD)

    s0 = edge_index[0].astype(jnp.int32)
    d0 = edge_index[1].astype(jnp.int32)
    loop = jnp.arange(N_, dtype=jnp.int32)
    src_all = jnp.concatenate([s0, loop])
    dst_all = jnp.concatenate([d0, loop])
    cmb_all = jnp.concatenate([
        edge_type.astype(jnp.int32) * 16 + nt_flat[s0] * 4 + nt_flat[d0],
        _ET * 16 + nt_flat * 5])
    padv = jnp.full((E_pad - E_tot,), -1, jnp.int32)
    src_all = jnp.concatenate([src_all, padv])
    dst_all = jnp.concatenate([dst_all, padv])
    cmb_all = jnp.concatenate([cmb_all, padv])
    src_col = src_all[:, None]
    dst_col = dst_all[:, None]
    dst_row = dst_all[None, :]
    cmb_col = cmb_all[:, None]

    eb_spec = pl.BlockSpec((_EB, 1), lambda i: (i, 0))
    full = lambda shp: pl.BlockSpec(shp, lambda i: (0, 0))

    _X = X
    for l in range(2):
        lp = p['layers'][l]
        xc = jnp.concatenate([_X, nfe], axis=1)
        Q = (xc @ lp['qW'].T + lp['qb']) / np.sqrt(_DPH)
        Kn = xc @ lp['kW'][:, :2 * _HID].T
        Mn = xc @ lp['mW'][:, :2 * _HID].T
        KE = tab_p @ lp['kW'][:, 2 * _HID:].T + lp['kb']
        ME = tab_p @ lp['mW'][:, 2 * _HID:].T + lp['mb']
        Qp, Kp, Mp = _pad_rows(Q, _NPAD), _pad_rows(Kn, _NPAD), _pad_rows(Mn, _NPAD)

        sc, smax = pl.pallas_call(
            _scores_kernel,
            grid=(nblk,),
            in_specs=[eb_spec, eb_spec, eb_spec,
                      full((_NPAD, _HID)), full((_NPAD, _HID)),
                      full((_CPAD, _HID))],
            out_specs=[pl.BlockSpec((_EB, _HEADS), lambda i: (i, 0)),
                       full((8, _NPAD))],
            out_shape=[jax.ShapeDtypeStruct((E_pad, _HEADS), jnp.float32),
                       jax.ShapeDtypeStruct((8, _NPAD), jnp.float32)],
        )(src_col, dst_col, cmb_col, Qp, Kp, KE)

        ex, den = pl.pallas_call(
            _softmax_kernel,
            grid=(nblk,),
            in_specs=[eb_spec,
                      pl.BlockSpec((_EB, _HEADS), lambda i: (i, 0)),
                      full((8, _NPAD))],
            out_specs=[pl.BlockSpec((_EB, _HEADS), lambda i: (i, 0)),
                       full((8, _NPAD))],
            out_shape=[jax.ShapeDtypeStruct((E_pad, _HEADS), jnp.float32),
                       jax.ShapeDtypeStruct((8, _NPAD), jnp.float32)],
        )(src_col, sc, smax)

        aggr = pl.pallas_call(
            _aggr_kernel,
            grid=(nblk,),
            in_specs=[eb_spec, pl.BlockSpec((1, _EB), lambda i: (0, i)),
                      eb_spec,
                      pl.BlockSpec((_EB, _HEADS), lambda i: (i, 0)),
                      full((8, _NPAD)),
                      full((_NPAD, _HID)), full((_CPAD, _HID))],
            out_specs=full((_NPAD, _HID)),
            out_shape=jax.ShapeDtypeStruct((_NPAD, _HID), jnp.float32),
        )(src_col, dst_row, cmb_col, ex, den, Mp, ME)

        h2 = aggr[:N_] @ lp['p1W'].T + lp['p1b']
        h2 = (h2 / np.sqrt(1.0 + 1e-5)) * lp['g'] + lp['be']
        h2 = jax.nn.relu(h2)
        _X = _gelu(h2 @ lp['p2W'].T + lp['p2b'])

    Xout = _X.reshape(bs, nn_, _HID)
    return _gelu(H @ p['VhW'].T + p['Vhb'] + Xout @ p['VxW'].T + p['Vxb'])


# speedup vs baseline: 1.0624x; 1.0624x over previous
import numpy as np
import jax
import jax.numpy as jnp
from jax.experimental import pallas as pl

_HID = 128
_HEADS = 4
_DPH = _HID // _HEADS
_NT = 4
_ET = 38

_EB = 1024
_NC = 1024
_NPAD = 10240
_CPAD = 640
_NCHUNKS = _NPAD // _NC


def _scores_kernel(src_ref, dst_ref, cmb_ref, q_ref, k_ref, ke_ref,
                   sc_ref, smax_ref):
    @pl.when(pl.program_id(0) == 0)
    def _init():
        smax_ref[...] = jnp.full(smax_ref.shape, -jnp.inf, jnp.float32)

    src = src_ref[...]
    dst = dst_ref[...]
    cmb = cmb_ref[...]

    qs = jnp.zeros((_EB, _HID), jnp.float32)
    kd = jnp.zeros((_EB, _HID), jnp.float32)
    for c in range(_NCHUNKS):
        col = jax.lax.broadcasted_iota(jnp.int32, (_EB, _NC), 1) + c * _NC
        ms = (src == col).astype(jnp.float32)
        md = (dst == col).astype(jnp.float32)
        qs = qs + jax.lax.dot_general(
            ms, q_ref[c * _NC:(c + 1) * _NC, :],
            (((1,), (0,)), ((), ())), preferred_element_type=jnp.float32)
        kd = kd + jax.lax.dot_general(
            md, k_ref[c * _NC:(c + 1) * _NC, :],
            (((1,), (0,)), ((), ())), preferred_element_type=jnp.float32)
    ccol = jax.lax.broadcasted_iota(jnp.int32, (_EB, _CPAD), 1)
    mc = (cmb == ccol).astype(jnp.float32)
    kec = jax.lax.dot_general(
        mc, ke_ref[...], (((1,), (0,)), ((), ())),
        preferred_element_type=jnp.float32)

    prod = qs * (kd + kec)
    sc = jnp.concatenate(
        [jnp.sum(prod[:, h * _DPH:(h + 1) * _DPH], axis=1, keepdims=True)
         for h in range(_HEADS)], axis=1)
    sc_ref[...] = sc

    for c in range(_NCHUNKS):
        col = jax.lax.broadcasted_iota(jnp.int32, (_EB, _NC), 1) + c * _NC
        msb = (src == col)
        for h in range(_HEADS):
            v = jnp.where(msb, sc[:, h:h + 1], -jnp.inf)
            u = jnp.max(v, axis=0, keepdims=True)
            cur = smax_ref[h:h + 1, c * _NC:(c + 1) * _NC]
            smax_ref[h:h + 1, c * _NC:(c + 1) * _NC] = jnp.maximum(cur, u)


def _softmax_kernel(src_ref, sc_ref, smax_ref, ex_ref, den_ref):
    @pl.when(pl.program_id(0) == 0)
    def _init():
        den_ref[...] = jnp.zeros(den_ref.shape, jnp.float32)

    src = src_ref[...]
    sc = sc_ref[...]
    smg = [jnp.zeros((_EB, 1), jnp.float32) for _ in range(_HEADS)]
    for c in range(_NCHUNKS):
        col = jax.lax.broadcasted_iota(jnp.int32, (_EB, _NC), 1) + c * _NC
        ms = (src == col).astype(jnp.float32)
        for h in range(_HEADS):
            row = smax_ref[h:h + 1, c * _NC:(c + 1) * _NC]
            smg[h] = smg[h] + jnp.sum(ms * row, axis=1, keepdims=True)
    smaxg = jnp.concatenate(smg, axis=1)
    ex = jnp.exp(sc - smaxg)
    ex_ref[...] = ex
    for c in range(_NCHUNKS):
        col = jax.lax.broadcasted_iota(jnp.int32, (_EB, _NC), 1) + c * _NC
        ms = (src == col).astype(jnp.float32)
        for h in range(_HEADS):
            contrib = jnp.sum(ms * ex[:, h:h + 1], axis=0, keepdims=True)
            den_ref[h:h + 1, c * _NC:(c + 1) * _NC] = (
                den_ref[h:h + 1, c * _NC:(c + 1) * _NC] + contrib)
        cnt = jnp.sum(ms, axis=0, keepdims=True)
        den_ref[4:5, c * _NC:(c + 1) * _NC] = (
            den_ref[4:5, c * _NC:(c + 1) * _NC] + cnt)


def _aggr_kernel(src_ref, dstr_ref, cmb_ref, ex_ref, den_ref, m_ref, me_ref,
                 out_ref):
    @pl.when(pl.program_id(0) == 0)
    def _init():
        out_ref[...] = jnp.zeros(out_ref.shape, jnp.float32)

    src = src_ref[...]
    dstr = dstr_ref[...]
    cmb = cmb_ref[...]
    ex = ex_ref[...]

    msrc = jnp.zeros((_EB, _HID), jnp.float32)
    dg = [jnp.zeros((_EB, 1), jnp.float32) for _ in range(5)]
    for c in range(_NCHUNKS):
        col = jax.lax.broadcasted_iota(jnp.int32, (_EB, _NC), 1) + c * _NC
        ms = (src == col).astype(jnp.float32)
        msrc = msrc + jax.lax.dot_general(
            ms.astype(jnp.bfloat16),
            m_ref[c * _NC:(c + 1) * _NC, :].astype(jnp.bfloat16),
            (((1,), (0,)), ((), ())), preferred_element_type=jnp.float32)
        for j in range(5):
            row = den_ref[j:j + 1, c * _NC:(c + 1) * _NC]
            dg[j] = dg[j] + jnp.sum(ms * row, axis=1, keepdims=True)
    deng = jnp.concatenate(dg[:4], axis=1)
    cntg = dg[4]
    alpha = ex / (deng + 1e-16) * cntg

    ccol = jax.lax.broadcasted_iota(jnp.int32, (_EB, _CPAD), 1)
    mc = (cmb == ccol).astype(jnp.float32)
    mec = jax.lax.dot_general(
        mc, me_ref[...], (((1,), (0,)), ((), ())),
        preferred_element_type=jnp.float32)

    r = jax.lax.broadcasted_iota(jnp.int32, (_HEADS, _HID), 0)
    l = jax.lax.broadcasted_iota(jnp.int32, (_HEADS, _HID), 1) // _DPH
    he = (r == l).astype(jnp.float32)
    alpha128 = jax.lax.dot_general(
        alpha, he, (((1,), (0,)), ((), ())),
        preferred_element_type=jnp.float32)
    m = (msrc + mec) * alpha128

    for c in range(_NCHUNKS):
        rowi = jax.lax.broadcasted_iota(jnp.int32, (_NC, _EB), 0) + c * _NC
        mdT = (rowi == dstr).astype(jnp.bfloat16)
        acc = jax.lax.dot_general(
            mdT, m.astype(jnp.bfloat16), (((1,), (0,)), ((), ())),
            preferred_element_type=jnp.float32)
        out_ref[c * _NC:(c + 1) * _NC, :] = (
            out_ref[c * _NC:(c + 1) * _NC, :] + acc)


def _gelu(x):
    return jax.nn.gelu(x, approximate=True)


def _pad_rows(x, n):
    return jnp.pad(x, ((0, n - x.shape[0]), (0, 0)))


def kernel(H, node_score, params, edge_index, edge_type, node_type):
    p = params
    bs, nn_ = node_type.shape
    N_ = bs * nn_
    E = edge_index.shape[1]
    E_tot = E + N_
    nblk = -(-E_tot // _EB)
    E_pad = nblk * _EB

    T = jax.nn.one_hot(node_type, _NT, dtype=jnp.float32)
    node_type_emb = _gelu(T @ p['ent_W'].T + p['ent_b'])
    js = jnp.power(1.1, jnp.arange(_HID // 2, dtype=jnp.float32))[None, None, :]
    node_score_emb = _gelu(jnp.sin(js * node_score) @ p['esc_W'].T + p['esc_b'])
    X = H.reshape(N_, -1)
    nt_flat = node_type.reshape(-1)
    nfe = jnp.concatenate([node_type_emb, node_score_emb], axis=2).reshape(N_, -1)

    cid = jnp.arange(_ET + 1, dtype=jnp.int32)
    cc = (cid[:, None, None] * 16 + jnp.arange(_NT, dtype=jnp.int32)[None, :, None] * 4
          + jnp.arange(_NT, dtype=jnp.int32)[None, None, :]).reshape(-1)
    ev = jax.nn.one_hot(cc // 16, _ET + 1, dtype=jnp.float32)
    ha = jax.nn.one_hot((cc // 4) % 4, _NT, dtype=jnp.float32)
    hb = jax.nn.one_hot(cc % 4, _NT, dtype=jnp.float32)
    feat = jnp.concatenate([ev, ha, hb], axis=1)
    h1 = feat @ p['ee_W1'].T + p['ee_b1']
    h1 = (h1 / np.sqrt(1.0 + 1e-5)) * p['ee_g'] + p['ee_be']
    tab = jax.nn.relu(h1) @ p['ee_W2'].T + p['ee_b2']
    tab_p = _pad_rows(tab, _CPAD)

    s0 = edge_index[0].astype(jnp.int32)
    d0 = edge_index[1].astype(jnp.int32)
    loop = jnp.arange(N_, dtype=jnp.int32)
    src_all = jnp.concatenate([s0, loop])
    dst_all = jnp.concatenate([d0, loop])
    cmb_all = jnp.concatenate([
        edge_type.astype(jnp.int32) * 16 + nt_flat[s0] * 4 + nt_flat[d0],
        _ET * 16 + nt_flat * 5])
    padv = jnp.full((E_pad - E_tot,), -1, jnp.int32)
    src_all = jnp.concatenate([src_all, padv])
    dst_all = jnp.concatenate([dst_all, padv])
    cmb_all = jnp.concatenate([cmb_all, padv])
    src_col = src_all[:, None]
    dst_col = dst_all[:, None]
    dst_row = dst_all[None, :]
    cmb_col = cmb_all[:, None]

    eb_spec = pl.BlockSpec((_EB, 1), lambda i: (i, 0))
    full = lambda shp: pl.BlockSpec(shp, lambda i: (0, 0))

    _X = X
    for l in range(2):
        lp = p['layers'][l]
        xc = jnp.concatenate([_X, nfe], axis=1)
        Q = (xc @ lp['qW'].T + lp['qb']) / np.sqrt(_DPH)
        Kn = xc @ lp['kW'][:, :2 * _HID].T
        Mn = xc @ lp['mW'][:, :2 * _HID].T
        KE = tab_p @ lp['kW'][:, 2 * _HID:].T + lp['kb']
        ME = tab_p @ lp['mW'][:, 2 * _HID:].T + lp['mb']
        Qp, Kp, Mp = _pad_rows(Q, _NPAD), _pad_rows(Kn, _NPAD), _pad_rows(Mn, _NPAD)

        sc, smax = pl.pallas_call(
            _scores_kernel,
            grid=(nblk,),
            in_specs=[eb_spec, eb_spec, eb_spec,
                      full((_NPAD, _HID)), full((_NPAD, _HID)),
                      full((_CPAD, _HID))],
            out_specs=[pl.BlockSpec((_EB, _HEADS), lambda i: (i, 0)),
                       full((8, _NPAD))],
            out_shape=[jax.ShapeDtypeStruct((E_pad, _HEADS), jnp.float32),
                       jax.ShapeDtypeStruct((8, _NPAD), jnp.float32)],
        )(src_col, dst_col, cmb_col, Qp, Kp, KE)

        ex, den = pl.pallas_call(
            _softmax_kernel,
            grid=(nblk,),
            in_specs=[eb_spec,
                      pl.BlockSpec((_EB, _HEADS), lambda i: (i, 0)),
                      full((8, _NPAD))],
            out_specs=[pl.BlockSpec((_EB, _HEADS), lambda i: (i, 0)),
                       full((8, _NPAD))],
            out_shape=[jax.ShapeDtypeStruct((E_pad, _HEADS), jnp.float32),
                       jax.ShapeDtypeStruct((8, _NPAD), jnp.float32)],
        )(src_col, sc, smax)

        aggr = pl.pallas_call(
            _aggr_kernel,
            grid=(nblk,),
            in_specs=[eb_spec, pl.BlockSpec((1, _EB), lambda i: (0, i)),
                      eb_spec,
                      pl.BlockSpec((_EB, _HEADS), lambda i: (i, 0)),
                      full((8, _NPAD)),
                      full((_NPAD, _HID)), full((_CPAD, _HID))],
            out_specs=full((_NPAD, _HID)),
            out_shape=jax.ShapeDtypeStruct((_NPAD, _HID), jnp.float32),
        )(src_col, dst_row, cmb_col, ex, den, Mp, ME)

        h2 = aggr[:N_] @ lp['p1W'].T + lp['p1b']
        h2 = (h2 / np.sqrt(1.0 + 1e-5)) * lp['g'] + lp['be']
        h2 = jax.nn.relu(h2)
        _X = _gelu(h2 @ lp['p2W'].T + lp['p2b'])

    Xout = _X.reshape(bs, nn_, _HID)
    return _gelu(H @ p['VhW'].T + p['Vhb'] + Xout @ p['VxW'].T + p['Vxb'])
